# baseline (speedup 1.0000x reference)
import functools

import jax
import jax.numpy as jnp
from jax.experimental import pallas as pl
from jax.experimental.pallas import tpu as pltpu

N = 100000
E = 3200000
OUT_C = 16


def _edge_conv(h, src, dst, Wa, ba, Wb, bb):
    h_i = h[dst]
    h_j = h[src]
    tmp = jnp.concatenate([h_i, h_j - h_i], axis=1)
    m = jnp.maximum(tmp @ Wa + ba, 0.0) @ Wb + bb
    agg = jax.ops.segment_max(m, dst, num_segments=N)
    agg = jnp.where(jnp.isneginf(agg), 0.0, agg)
    return agg


def _final_body(x2_ref, x4_ref, x6_ref, wf_ref, bf_ref, o_ref):
    x7 = jnp.concatenate([x2_ref[...], x4_ref[...], x6_ref[...]], axis=1)
    cols = jnp.arange(48)[:, None]
    ks = jnp.arange(OUT_C)[None, :]
    sel = [(cols == 3 * ks + c).astype(jnp.float32) for c in range(3)]
    x8 = jnp.maximum(jnp.maximum(x7 @ sel[0], x7 @ sel[1]), x7 @ sel[2])
    wf = wf_ref[...].reshape(1, OUT_C)
    o_ref[...] = jnp.sum(x8 * wf, axis=1, keepdims=True) + bf_ref[0]


def _final_stage(x2, x4, x6, Wf, bf):
    BR = 2000
    grid = (N // BR,)
    return pl.pallas_call(
        _final_body,
        grid=grid,
        in_specs=[
            pl.BlockSpec((BR, OUT_C), lambda i: (i, 0)),
            pl.BlockSpec((BR, OUT_C), lambda i: (i, 0)),
            pl.BlockSpec((BR, OUT_C), lambda i: (i, 0)),
            pl.BlockSpec((OUT_C, 1), lambda i: (0, 0)),
            pl.BlockSpec(memory_space=pltpu.SMEM),
        ],
        out_specs=pl.BlockSpec((BR, 1), lambda i: (i, 0)),
        out_shape=jax.ShapeDtypeStruct((N, 1), jnp.float32),
    )(x2, x4, x6, Wf, bf)


def kernel(x, edge_index, W1a, b1a, W1b, b1b, W2a, b2a, W2b, b2b, W3a, b3a, W3b, b3b, Wf, bf):
    src = edge_index[0]
    dst = edge_index[1]
    x1 = _edge_conv(x, src, dst, W1a, b1a, W1b, b1b)
    x2 = jax.nn.relu(x1)
    x3 = _edge_conv(x2, src, dst, W2a, b2a, W2b, b2b)
    x4 = jax.nn.relu(x3)
    x5 = _edge_conv(x4, src, dst, W3a, b3a, W3b, b3b)
    x6 = jax.nn.relu(x5)
    return _final_stage(x2, x4, x6, Wf, bf)


# speedup vs baseline: 2.5456x; 2.5456x over previous
import functools

import jax
import jax.numpy as jnp
from jax import lax
from jax.experimental import pallas as pl
from jax.experimental.pallas import tpu as pltpu
from jax.experimental.pallas import tpu_sc as plsc

N = 100000
E = 3200000
C = 16

NW = 32
NBIN = 32
BIN_N = N // NBIN
LCAP = 288
CAP = 16 * LCAP
BLK = 128
NBLK = CAP // BLK
NSUP = NBLK // 4
ECAP = NW * NBIN * CAP
N_PAD = 102400
ACC_ROWS = 3136
CHUNK = E // NW
QUINT = 10000
NQ = CHUNK // QUINT
BPP = 8
NPASS = NBIN // BPP

_MESH = plsc.VectorSubcoreMesh(
    core_axis_name="c", subcore_axis_name="s", num_cores=2, num_subcores=16)


def _wid():
    return lax.axis_index("s") * 2 + lax.axis_index("c")


def _bin_body(src_hbm, dst_hbm, srcp_hbm, dstp_hbm,
              in_src, in_dst, reg_src, reg_dst):
    wid = _wid()
    ebase = wid * CHUNK
    zeros16 = jnp.zeros((16,), jnp.int32)
    lanes = lax.iota(jnp.int32, 16) * LCAP
    for p in range(NPASS):

        def quint_body(q, offs, p=p):
            qs = ebase + q * QUINT
            pltpu.sync_copy(src_hbm.at[pl.ds(qs, QUINT)], in_src)
            pltpu.sync_copy(dst_hbm.at[pl.ds(qs, QUINT)], in_dst)

            def vec_body(v, offs):
                dvec = in_dst[pl.ds(v * 16, 16)]
                svec = in_src[pl.ds(v * 16, 16)]
                new_offs = []
                for bl in range(BPP):
                    lo = (p * BPP + bl) * BIN_N
                    off = offs[bl]
                    msk = (dvec >= lo) & (dvec < lo + BIN_N) & (off < LCAP)
                    pos = (bl * CAP + lanes) + off
                    plsc.store_scatter(reg_dst, [pos], dvec, mask=msk)
                    plsc.store_scatter(reg_src, [pos], svec, mask=msk)
                    new_offs.append(off + msk.astype(jnp.int32))
                return tuple(new_offs)

            return lax.fori_loop(0, QUINT // 16, vec_body, offs)

        offs = lax.fori_loop(0, NQ, quint_body,
                             tuple(zeros16 for _ in range(BPP)))

        for bl in range(BPP):
            gb = p * BPP + bl
            dummy = jnp.full((16,), (gb + 1) * BIN_N, jnp.int32)

            def pad_body(i, off, bl=bl, dummy=dummy):
                msk = off < LCAP
                pos = (bl * CAP + lanes) + off
                plsc.store_scatter(reg_dst, [pos], dummy, mask=msk)
                plsc.store_scatter(reg_src, [pos], zeros16, mask=msk)
                return off + msk.astype(jnp.int32)

            lax.fori_loop(0, LCAP, pad_body, offs[bl])
            rbase = (wid * NBIN + gb) * CAP
            pltpu.sync_copy(reg_src.at[pl.ds(bl * CAP, CAP)],
                            srcp_hbm.at[pl.ds(rbase, CAP)])
            pltpu.sync_copy(reg_dst.at[pl.ds(bl * CAP, CAP)],
                            dstp_hbm.at[pl.ds(rbase, CAP)])


_bin_call = pl.kernel(
    _bin_body,
    out_type=(jax.ShapeDtypeStruct((ECAP,), jnp.int32),
              jax.ShapeDtypeStruct((ECAP,), jnp.int32)),
    mesh=_MESH,
    compiler_params=pltpu.CompilerParams(needs_layout_passes=False, use_tc_tiling_on_sc=False),
    scratch_types=[
        pltpu.VMEM((QUINT,), jnp.int32),
        pltpu.VMEM((QUINT,), jnp.int32),
        pltpu.VMEM((BPP * CAP,), jnp.int32),
        pltpu.VMEM((BPP * CAP,), jnp.int32),
    ],
)


def _gather_body(g_hbm, u_hbm, srcp_hbm, dstp_hbm, p_hbm,
                 idx_s, idx_d, gbuf, ubuf, pbuf,
                 sg0, sg1, sg2, sg3, su0, su1, su2, su3, so0, so1, so2, so3):
    wid = _wid()
    sg = [sg0, sg1, sg2, sg3]
    su = [su0, su1, su2, su3]
    so = [so0, so1, so2, so3]

    def issue(pp, blk):
        pltpu.async_copy(
            g_hbm.at[idx_d.at[pl.ds(blk * BLK, BLK)]], gbuf.at[pp], sg[pp])
        pltpu.async_copy(
            u_hbm.at[idx_s.at[pl.ds(blk * BLK, BLK)]], ubuf.at[pp], su[pp])

    def drain(buf, sem):
        pltpu.make_async_copy(g_hbm.at[pl.ds(0, BLK)], buf, sem).wait()

    def region_body(r, _):
        rbase = (wid * NBIN + r) * CAP
        pltpu.sync_copy(srcp_hbm.at[pl.ds(rbase, CAP)], idx_s)
        pltpu.sync_copy(dstp_hbm.at[pl.ds(rbase, CAP)], idx_d)
        for pp in range(4):
            issue(pp, pp)

        def super_body(s, _):
            for pp in range(4):
                b = s * 4 + pp
                drain(gbuf.at[pp], sg[pp])
                drain(ubuf.at[pp], su[pp])

                @pl.when((s > 0) | (r > 0))
                def _():
                    drain(pbuf.at[pp], so[pp])

                def add_body(i, _, pp=pp):
                    pbuf[pp, i] = gbuf[pp, i] + ubuf[pp, i]
                    return 0

                lax.fori_loop(0, BLK, add_body, 0)
                pltpu.async_copy(
                    pbuf.at[pp], p_hbm.at[pl.ds(rbase + b * BLK, BLK)], so[pp])

                @pl.when(s < NSUP - 1)
                def _(pp=pp, b=b):
                    issue(pp, b + 4)

            return 0

        lax.fori_loop(0, NSUP, super_body, 0)
        return 0

    lax.fori_loop(0, NBIN, region_body, 0)
    for pp in range(4):
        drain(pbuf.at[pp], so[pp])


_gather_call = pl.kernel(
    _gather_body,
    out_type=jax.ShapeDtypeStruct((ECAP, C), jnp.float32),
    mesh=_MESH,
    compiler_params=pltpu.CompilerParams(needs_layout_passes=False, use_tc_tiling_on_sc=False),
    scratch_types=(
        [pltpu.VMEM((CAP,), jnp.int32)] * 2
        + [pltpu.VMEM((4, BLK, C), jnp.float32)] * 3
        + [pltpu.SemaphoreType.DMA] * 12
    ),
)


def _scatter_body(m_hbm, dstp_hbm, h_hbm, acc, dstv, mbuf, sm0, sm1, sm2, sm3):
    tid = _wid()
    base_node = tid * BIN_N
    sm = [sm0, sm1, sm2, sm3]

    def z_body(i, _):
        acc[i] = jnp.zeros((C,), jnp.float32)
        return 0

    lax.fori_loop(0, ACC_ROWS, z_body, 0)

    def region_body(r, _):
        rbase = (r * NBIN + tid) * CAP
        pltpu.sync_copy(dstp_hbm.at[pl.ds(rbase, CAP)], dstv)
        for pp in range(4):
            pltpu.async_copy(
                m_hbm.at[pl.ds(rbase + pp * BLK, BLK)], mbuf.at[pp], sm[pp])

        def super_body(s, _):
            for pp in range(4):
                b = s * 4 + pp
                pltpu.make_async_copy(
                    m_hbm.at[pl.ds(0, BLK)], mbuf.at[pp], sm[pp]).wait()

                def grp_body(jg, _, pp=pp):
                    dvec = dstv[pl.ds(b * BLK + jg * 16, 16)] - base_node
                    for i in range(16):
                        d = dvec[i]
                        acc[d] = jnp.maximum(acc[d], mbuf[pp, jg * 16 + i])
                    return 0

                lax.fori_loop(0, BLK // 16, grp_body, 0)

                @pl.when(s < NSUP - 1)
                def _(pp=pp, b=b):
                    pltpu.async_copy(
                        m_hbm.at[pl.ds(rbase + (b + 4) * BLK, BLK)],
                        mbuf.at[pp], sm[pp])

            return 0

        lax.fori_loop(0, NSUP, super_body, 0)
        return 0

    lax.fori_loop(0, NBIN, region_body, 0)
    pltpu.sync_copy(acc.at[pl.ds(0, BIN_N)],
                    h_hbm.at[pl.ds(base_node, BIN_N)])


_scatter_call = pl.kernel(
    _scatter_body,
    out_type=jax.ShapeDtypeStruct((N_PAD, C), jnp.float32),
    mesh=_MESH,
    compiler_params=pltpu.CompilerParams(needs_layout_passes=False, use_tc_tiling_on_sc=False),
    scratch_types=(
        [pltpu.VMEM((ACC_ROWS, C), jnp.float32),
         pltpu.VMEM((CAP,), jnp.int32),
         pltpu.VMEM((4, BLK, C), jnp.float32)]
        + [pltpu.SemaphoreType.DMA] * 4
    ),
)


def _pre_body(h_ref, wd_ref, wb_ref, ba_ref, g_ref, u_ref):
    h = h_ref[...]
    g_ref[...] = h @ wd_ref[...]
    u_ref[...] = h @ wb_ref[...] + ba_ref[...]


def _pre_stage(h, Wd, Wbot, ba):
    BR = 2048
    return pl.pallas_call(
        _pre_body,
        grid=(N_PAD // BR,),
        in_specs=[
            pl.BlockSpec((BR, C), lambda i: (i, 0)),
            pl.BlockSpec((C, C), lambda i: (0, 0)),
            pl.BlockSpec((C, C), lambda i: (0, 0)),
            pl.BlockSpec((1, C), lambda i: (0, 0)),
        ],
        out_specs=[pl.BlockSpec((BR, C), lambda i: (i, 0))] * 2,
        out_shape=[jax.ShapeDtypeStruct((N_PAD, C), jnp.float32)] * 2,
    )(h, Wd, Wbot, ba.reshape(1, C))


def _mlp_body(p_ref, wb_ref, bb_ref, m_ref):
    m_ref[...] = jnp.maximum(p_ref[...], 0.0) @ wb_ref[...] + bb_ref[...]


def _mlp_stage(P, Wb, bb):
    BR = 8192
    return pl.pallas_call(
        _mlp_body,
        grid=(ECAP // BR,),
        in_specs=[
            pl.BlockSpec((BR, C), lambda i: (i, 0)),
            pl.BlockSpec((C, C), lambda i: (0, 0)),
            pl.BlockSpec((1, C), lambda i: (0, 0)),
        ],
        out_specs=pl.BlockSpec((BR, C), lambda i: (i, 0)),
        out_shape=jax.ShapeDtypeStruct((ECAP, C), jnp.float32),
    )(P, Wb, bb.reshape(1, C))


def _final_body(x2_ref, x4_ref, x6_ref, wf_ref, bf_ref, o_ref):
    x7 = jnp.concatenate([x2_ref[...], x4_ref[...], x6_ref[...]], axis=1)
    cols = jnp.arange(48)[:, None]
    ks = jnp.arange(C)[None, :]
    sel = [(cols == 3 * ks + c).astype(jnp.float32) for c in range(3)]
    x8 = jnp.maximum(jnp.maximum(x7 @ sel[0], x7 @ sel[1]), x7 @ sel[2])
    wf = wf_ref[...].reshape(1, C)
    o_ref[...] = jnp.sum(x8 * wf, axis=1, keepdims=True) + bf_ref[0]


def _final_stage(x2, x4, x6, Wf, bf):
    BR = 2000
    return pl.pallas_call(
        _final_body,
        grid=(N // BR,),
        in_specs=[
            pl.BlockSpec((BR, C), lambda i: (i, 0)),
            pl.BlockSpec((BR, C), lambda i: (i, 0)),
            pl.BlockSpec((BR, C), lambda i: (i, 0)),
            pl.BlockSpec((C, 1), lambda i: (0, 0)),
            pl.BlockSpec(memory_space=pltpu.SMEM),
        ],
        out_specs=pl.BlockSpec((BR, 1), lambda i: (i, 0)),
        out_shape=jax.ShapeDtypeStruct((N, 1), jnp.float32),
    )(x2, x4, x6, Wf, bf)


def kernel(x, edge_index, W1a, b1a, W1b, b1b, W2a, b2a, W2b, b2b,
           W3a, b3a, W3b, b3b, Wf, bf):
    src = edge_index[0]
    dst = edge_index[1]
    srcp, dstp = _bin_call(src, dst)

    W1d = jnp.zeros((C, C), jnp.float32).at[0:3].set(W1a[0:3] - W1a[3:6])
    W1bot = jnp.zeros((C, C), jnp.float32).at[0:3].set(W1a[3:6])
    x_pad = jnp.zeros((N_PAD, C), jnp.float32).at[:N, :3].set(x)

    layers = [
        (W1d, W1bot, b1a, W1b, b1b),
        (W2a[:C] - W2a[C:], W2a[C:], b2a, W2b, b2b),
        (W3a[:C] - W3a[C:], W3a[C:], b3a, W3b, b3b),
    ]
    h = x_pad
    hs = []
    for (Wd, Wbot, ba, Wb, bb) in layers:
        g, u = _pre_stage(h, Wd, Wbot, ba)
        P = _gather_call(g, u, srcp, dstp)
        m = _mlp_stage(P, Wb, bb)
        h = _scatter_call(m, dstp)
        hs.append(h)

    return _final_stage(hs[0], hs[1], hs[2], Wf, bf)
